# baseline (speedup 1.0000x reference)
import functools

import jax
import jax.numpy as jnp
from jax import lax
from jax.experimental import pallas as pl
from jax.experimental.pallas import tpu as pltpu
from jax.experimental.pallas import tpu_sc as plsc

_NUM_CORES = 2
_NUM_SUBCORES = 16
_NW = _NUM_CORES * _NUM_SUBCORES
_NBUF = 4


def _build(batch: int, seq_len: int, value_dim: int):
    batches_per_w = batch // _NW
    idx_per_w = batches_per_w * seq_len
    mesh = plsc.VectorSubcoreMesh(core_axis_name="c", subcore_axis_name="s")

    row_buf = pltpu.VMEM((seq_len, value_dim), jnp.float32)

    @functools.partial(
        pl.kernel,
        mesh=mesh,
        compiler_params=pltpu.CompilerParams(use_tc_tiling_on_sc=False),
        out_type=jax.ShapeDtypeStruct(
            (batch, seq_len, value_dim), jnp.float32
        ),
        scratch_types=(
            [pltpu.VMEM((idx_per_w,), jnp.int32)]
            + [row_buf] * _NBUF
            + [pltpu.SemaphoreType.DMA] * (2 * _NBUF)
        ),
    )
    def gather_kernel(table_hbm, idx_hbm, out_hbm, idx_v, *bufs_and_sems):
        rows = bufs_and_sems[:_NBUF]
        gsems = bufs_and_sems[_NBUF : 2 * _NBUF]
        ssems = bufs_and_sems[2 * _NBUF :]

        wid = lax.axis_index("s") * _NUM_CORES + lax.axis_index("c")
        base_b = wid * batches_per_w
        pltpu.sync_copy(idx_hbm.at[pl.ds(base_b * seq_len, idx_per_w)], idx_v)

        def fire_gather(c, b):
            pltpu.async_copy(
                table_hbm.at[idx_v.at[pl.ds(c * seq_len, seq_len)]],
                rows[b],
                gsems[b],
            )

        def wait_gather(b):
            pltpu.make_async_copy(
                table_hbm.at[pl.ds(0, seq_len)], rows[b], gsems[b]
            ).wait()

        def fire_store(c, b):
            pltpu.async_copy(rows[b], out_hbm.at[base_b + c], ssems[b])

        def wait_store(b):
            pltpu.make_async_copy(
                rows[b], out_hbm.at[base_b], ssems[b]
            ).wait()

        for b in range(_NBUF):
            fire_gather(b, b)

        @pl.loop(0, batches_per_w, step=_NBUF)
        def _(c):
            for b in range(_NBUF):
                wait_gather(b)
                fire_store(c + b, b)

            @pl.when(c + _NBUF < batches_per_w)
            def _():
                for b in range(_NBUF):
                    wait_store(b)
                    fire_gather(c + _NBUF + b, b)

        for b in range(_NBUF):
            wait_store(b)

    return gather_kernel


def _retile_table(Wt):
    j_dim, i_dim = Wt.shape
    grid = (i_dim + 127) // 128

    q = jax.lax.broadcasted_iota(jnp.int32, (128, 128), 1)
    mr = jax.lax.broadcasted_iota(jnp.int32, (128, 128), 0)
    A_all = (q == 4 * (mr % 32) + mr // 32).astype(jnp.float32)
    cc = jax.lax.broadcasted_iota(jnp.int32, (32, 128), 1)
    jj = jax.lax.broadcasted_iota(jnp.int32, (32, 128), 0)
    C = [(cc == 32 * m + jj).astype(jnp.float32) for m in range(4)]
    C_all = jnp.concatenate(C, axis=0)

    def body(x_ref, a_ref, c_ref, o_ref):
        x = x_ref[...]
        d_all = jax.lax.dot_general(
            a_ref[...],
            x,
            (((1,), (1,)), ((), ())),
            precision=jax.lax.Precision.HIGHEST,
            preferred_element_type=jnp.float32,
        )
        acc = jnp.zeros((32, 4 * j_dim), jnp.float32)
        for m in range(4):
            acc = acc + jax.lax.dot_general(
                d_all[32 * m : 32 * (m + 1), :],
                c_ref[32 * m : 32 * (m + 1), :],
                (((1,), (0,)), ((), ())),
                precision=jax.lax.Precision.HIGHEST,
                preferred_element_type=jnp.float32,
            )
        o_ref[...] = acc

    return pl.pallas_call(
        body,
        grid=(grid,),
        in_specs=[
            pl.BlockSpec((j_dim, 128), lambda g: (0, g)),
            pl.BlockSpec((128, 128), lambda g: (0, 0)),
            pl.BlockSpec((128, 128), lambda g: (0, 0)),
        ],
        out_specs=pl.BlockSpec((32, 4 * j_dim), lambda g: (g, 0)),
        out_shape=jax.ShapeDtypeStruct(
            (i_dim // 4, 4 * j_dim), jnp.float32
        ),
    )(Wt, A_all, C_all)


def kernel(x, W):
    batch, seq_len = x.shape
    n_rows, value_dim = W.shape
    wide = _retile_table(jnp.swapaxes(W, 0, 1))
    table = wide.reshape(n_rows * value_dim).reshape(n_rows, value_dim)
    idx = x.reshape(batch * seq_len)
    return _build(batch, seq_len, value_dim)(table, idx)


# speedup vs baseline: 3.0814x; 3.0814x over previous
import functools

import jax
import jax.numpy as jnp
from jax import lax
from jax.experimental import pallas as pl
from jax.experimental.pallas import tpu as pltpu
from jax.experimental.pallas import tpu_sc as plsc

_NUM_CORES = 2
_NUM_SUBCORES = 16
_NW = _NUM_CORES * _NUM_SUBCORES
_NBUF = 4


def _build(batch: int, seq_len: int, value_dim: int):
    batches_per_w = batch // _NW
    idx_per_w = batches_per_w * seq_len
    mesh = plsc.VectorSubcoreMesh(core_axis_name="c", subcore_axis_name="s")

    row_buf = pltpu.VMEM((seq_len, value_dim), jnp.float32)

    @functools.partial(
        pl.kernel,
        mesh=mesh,
        compiler_params=pltpu.CompilerParams(use_tc_tiling_on_sc=False),
        out_type=jax.ShapeDtypeStruct(
            (batch, seq_len, value_dim), jnp.float32
        ),
        scratch_types=(
            [pltpu.VMEM((idx_per_w,), jnp.int32)]
            + [row_buf] * _NBUF
            + [pltpu.SemaphoreType.DMA] * (2 * _NBUF)
        ),
    )
    def gather_kernel(table_hbm, idx_hbm, out_hbm, idx_v, *bufs_and_sems):
        rows = bufs_and_sems[:_NBUF]
        gsems = bufs_and_sems[_NBUF : 2 * _NBUF]
        ssems = bufs_and_sems[2 * _NBUF :]

        wid = lax.axis_index("s") * _NUM_CORES + lax.axis_index("c")
        base_b = wid * batches_per_w
        pltpu.sync_copy(idx_hbm.at[pl.ds(base_b * seq_len, idx_per_w)], idx_v)

        def fire_gather(c, b):
            pltpu.async_copy(
                table_hbm.at[idx_v.at[pl.ds(c * seq_len, seq_len)]],
                rows[b],
                gsems[b],
            )

        def wait_gather(b):
            pltpu.make_async_copy(
                table_hbm.at[pl.ds(0, seq_len)], rows[b], gsems[b]
            ).wait()

        def fire_store(c, b):
            pltpu.async_copy(rows[b], out_hbm.at[base_b + c], ssems[b])

        def wait_store(b):
            pltpu.make_async_copy(
                rows[b], out_hbm.at[base_b], ssems[b]
            ).wait()

        for b in range(_NBUF):
            fire_gather(b, b)

        @pl.loop(0, batches_per_w, step=_NBUF)
        def _(c):
            for b in range(_NBUF):
                wait_gather(b)
                fire_store(c + b, b)

            @pl.when(c + _NBUF < batches_per_w)
            def _():
                for b in range(_NBUF):
                    wait_store(b)
                    fire_gather(c + _NBUF + b, b)

        for b in range(_NBUF):
            wait_store(b)

    return gather_kernel


def _retile_table(Wt):
    j_dim, i_dim = Wt.shape
    sub = 64
    step_lanes = 128 * sub
    grid = (i_dim + step_lanes - 1) // step_lanes

    q = jax.lax.broadcasted_iota(jnp.int32, (128, 128), 1)
    mr = jax.lax.broadcasted_iota(jnp.int32, (128, 128), 0)
    A_all = (q == 4 * (mr % 32) + mr // 32).astype(jnp.float32)
    cc = jax.lax.broadcasted_iota(jnp.int32, (32, 128), 1)
    jj = jax.lax.broadcasted_iota(jnp.int32, (32, 128), 0)
    C = [(cc == 32 * m + jj).astype(jnp.float32) for m in range(4)]
    C_all = jnp.concatenate(C, axis=0)

    def body(x_ref, a_ref, c_ref, o_ref):
        for k in range(sub):
            x = x_ref[:, 128 * k : 128 * (k + 1)]
            d_all = jax.lax.dot_general(
                a_ref[...],
                x,
                (((1,), (1,)), ((), ())),
                precision=jax.lax.Precision.HIGHEST,
                preferred_element_type=jnp.float32,
            )
            acc = jnp.zeros((32, 4 * j_dim), jnp.float32)
            for m in range(4):
                acc = acc + jax.lax.dot_general(
                    d_all[32 * m : 32 * (m + 1), :],
                    c_ref[32 * m : 32 * (m + 1), :],
                    (((1,), (0,)), ((), ())),
                    precision=jax.lax.Precision.HIGHEST,
                    preferred_element_type=jnp.float32,
                )
            o_ref[32 * k : 32 * (k + 1), :] = acc

    return pl.pallas_call(
        body,
        grid=(grid,),
        in_specs=[
            pl.BlockSpec((j_dim, step_lanes), lambda g: (0, g)),
            pl.BlockSpec((128, 128), lambda g: (0, 0)),
            pl.BlockSpec((128, 128), lambda g: (0, 0)),
        ],
        out_specs=pl.BlockSpec((32 * sub, 4 * j_dim), lambda g: (g, 0)),
        out_shape=jax.ShapeDtypeStruct(
            (i_dim // 4, 4 * j_dim), jnp.float32
        ),
    )(Wt, A_all, C_all)


def kernel(x, W):
    batch, seq_len = x.shape
    n_rows, value_dim = W.shape
    wide = _retile_table(jnp.swapaxes(W, 0, 1))
    table = wide.reshape(n_rows * value_dim).reshape(n_rows, value_dim)
    idx = x.reshape(batch * seq_len)
    return _build(batch, seq_len, value_dim)(table, idx)


# speedup vs baseline: 5.7286x; 1.8591x over previous
import functools

import jax
import jax.numpy as jnp
from jax import lax
from jax.experimental import pallas as pl
from jax.experimental.pallas import tpu as pltpu
from jax.experimental.pallas import tpu_sc as plsc

_NUM_CORES = 2
_NUM_SUBCORES = 16
_NW = _NUM_CORES * _NUM_SUBCORES
_CHUNK = 256
_NBUF = 4


def _build(num_indices: int, value_dim: int):
    b_per_w = num_indices // _NW
    n_chunks = b_per_w // _CHUNK
    assert n_chunks % _NBUF == 0 and n_chunks >= 2 * _NBUF
    mesh = plsc.VectorSubcoreMesh(core_axis_name="c", subcore_axis_name="s")

    row_buf = pltpu.VMEM((_CHUNK, value_dim), jnp.float32)

    @functools.partial(
        pl.kernel,
        mesh=mesh,
        compiler_params=pltpu.CompilerParams(use_tc_tiling_on_sc=False),
        out_type=jax.ShapeDtypeStruct((num_indices, value_dim), jnp.float32),
        scratch_types=(
            [pltpu.VMEM((b_per_w,), jnp.int32)]
            + [row_buf] * _NBUF
            + [pltpu.SemaphoreType.DMA] * (2 * _NBUF)
        ),
    )
    def gather_kernel(table_hbm, idx_hbm, out_hbm, idx_v, *bufs_and_sems):
        rows = bufs_and_sems[:_NBUF]
        gsems = bufs_and_sems[_NBUF : 2 * _NBUF]
        ssems = bufs_and_sems[2 * _NBUF :]

        wid = lax.axis_index("s") * _NUM_CORES + lax.axis_index("c")
        base = wid * b_per_w
        pltpu.sync_copy(idx_hbm.at[pl.ds(base, b_per_w)], idx_v)

        def fire_gather(c, b):
            pltpu.async_copy(
                table_hbm.at[idx_v.at[pl.ds(c * _CHUNK, _CHUNK)]],
                rows[b],
                gsems[b],
            )

        def wait_gather(b):
            pltpu.make_async_copy(
                table_hbm.at[pl.ds(0, _CHUNK)], rows[b], gsems[b]
            ).wait()

        def fire_store(c, b):
            pltpu.async_copy(
                rows[b], out_hbm.at[pl.ds(base + c * _CHUNK, _CHUNK)], ssems[b]
            )

        def wait_store(b):
            pltpu.make_async_copy(
                rows[b], out_hbm.at[pl.ds(base, _CHUNK)], ssems[b]
            ).wait()

        for b in range(_NBUF):
            fire_gather(b, b)

        @pl.loop(0, n_chunks, step=_NBUF)
        def _(c):
            for b in range(_NBUF):
                wait_gather(b)
                fire_store(c + b, b)

            @pl.when(c + _NBUF < n_chunks)
            def _():
                for b in range(_NBUF):
                    wait_store(b)
                    fire_gather(c + _NBUF + b, b)

        for b in range(_NBUF):
            wait_store(b)

    return gather_kernel


def kernel(x, W):
    batch, seq_len = x.shape
    num_indices = batch * seq_len
    idx = x.reshape(num_indices)
    out = _build(num_indices, W.shape[1])(W, idx)
    return out.reshape(batch, seq_len, W.shape[1])
